# baseline (speedup 1.0000x reference)
import functools

import jax
import jax.numpy as jnp
from jax import lax
from jax.experimental import pallas as pl
from jax.experimental.pallas import tpu as pltpu
from jax.experimental.pallas import tpu_sc as plsc

N_NODES = 10000
N_EDGES = 160000
MAX_Z = 94
N_ORB = 37
EMB = 16
D_OUT = N_ORB * EMB

LANES = 16


def _gather_zj_sc(z, idx_j):
    info = plsc.get_sparse_core_info()
    nc, ns = info.num_cores, info.num_subcores
    nw = nc * ns
    epw = N_EDGES // nw
    n_iters = (epw + LANES - 1) // LANES
    buf = n_iters * LANES + LANES

    mesh = plsc.VectorSubcoreMesh(core_axis_name="c", subcore_axis_name="s")

    @functools.partial(
        pl.kernel,
        mesh=mesh,
        compiler_params=pltpu.CompilerParams(needs_layout_passes=False),
        out_type=jax.ShapeDtypeStruct((N_EDGES,), jnp.int32),
        scratch_types=[
            pltpu.VMEM((N_NODES,), jnp.int32),
            pltpu.VMEM((buf,), jnp.int32),
            pltpu.VMEM((buf,), jnp.int32),
        ],
    )
    def zj_kernel(z_hbm, idx_hbm, zj_hbm, z_v, idx_v, out_v):
        wid = lax.axis_index("s") * nc + lax.axis_index("c")
        base = wid * epw
        pltpu.sync_copy(z_hbm, z_v)
        pltpu.sync_copy(idx_hbm.at[pl.ds(base, epw)], idx_v.at[pl.ds(0, epw)])
        idx_v[pl.ds(epw, LANES)] = jnp.zeros((LANES,), jnp.int32)

        def body(i, carry):
            idx16 = idx_v[pl.ds(i * LANES, LANES)]
            out_v[pl.ds(i * LANES, LANES)] = plsc.load_gather(z_v, [idx16])
            return carry

        lax.fori_loop(0, n_iters, body, 0)
        pltpu.sync_copy(out_v.at[pl.ds(0, epw)], zj_hbm.at[pl.ds(base, epw)])

    return zj_kernel(z, idx_j)


_BE = 6400
_NB = N_EDGES // _BE


def _expand_tc_body(zj_ref, val_ref, out_ref):
    zjb = jnp.broadcast_to(zj_ref[0], (MAX_Z, _BE))
    onehot_t = (zjb == lax.broadcasted_iota(jnp.int32, (MAX_Z, _BE), 0)).astype(
        jnp.bfloat16
    )
    m = lax.dot_general(
        val_ref[...].astype(jnp.bfloat16),
        onehot_t,
        dimension_numbers=(((0,), (0,)), ((), ())),
        preferred_element_type=jnp.float32,
    )
    out_ref[...] = jnp.broadcast_to(m[:, None, :], (N_ORB, EMB, _BE))


def _expand_tc(zj, valence):
    out_t = pl.pallas_call(
        _expand_tc_body,
        grid=(_NB,),
        in_specs=[
            pl.BlockSpec((1, 1, _BE), lambda i: (i, 0, 0)),
            pl.BlockSpec((MAX_Z, N_ORB), lambda i: (0, 0)),
        ],
        out_specs=pl.BlockSpec((N_ORB, EMB, _BE), lambda i: (0, 0, i)),
        out_shape=jax.ShapeDtypeStruct((N_ORB, EMB, N_EDGES), jnp.float32),
    )(zj.reshape(_NB, 1, _BE), valence)
    return jnp.transpose(out_t, (2, 0, 1))


def kernel(z, idx_j, valence):
    zj = _gather_zj_sc(z, idx_j)
    return _expand_tc(zj, valence)
